# baseline (speedup 1.0000x reference)
import functools

import jax
import jax.numpy as jnp
from jax import lax
from jax.experimental import pallas as pl
from jax.experimental.pallas import tpu as pltpu
from jax.experimental.pallas import tpu_sc as plsc

_NC = 2
_NS = 16
_NW = _NC * _NS
_C = 128
_G = 16


def _edge_aggregate(nodes_ft, src_kc, dst_kc, zeros_rows, n, d, k_chunks,
                    rows_per_tile):
    n_pad = _NS * rows_per_tile
    n_groups = k_chunks // _G
    mesh = plsc.VectorSubcoreMesh(core_axis_name="c", subcore_axis_name="s")

    @functools.partial(
        pl.kernel,
        out_type=jax.ShapeDtypeStruct((_NC, n_pad, d), jnp.float32),
        mesh=mesh,
        scratch_types=[
            pltpu.VMEM((3, _G, _C), jnp.int32),
            pltpu.VMEM((3, _G, _C), jnp.int32),
            pltpu.VMEM((2, _C // 2, 2 * d), jnp.float32),
            pltpu.VMEM_SHARED((n_pad, d), jnp.float32),
            pltpu.SemaphoreType.DMA,
            pltpu.SemaphoreType.DMA,
            pltpu.SemaphoreType.DMA,
        ],
    )
    def sck(nodes_hbm, nodes2_hbm, src_hbm, dst_hbm, zeros_hbm, out_hbm,
            src_g, dst_g, rows_v, acc_sh, sem0, sem1, semi):
        core = lax.axis_index("c")
        sub = lax.axis_index("s")
        wid = core * _NS + sub

        def idx_copy(hbm, ring, g, slot, sem):
            return pltpu.make_async_copy(
                hbm.at[wid].at[pl.ds(g * _G, _G)], ring.at[slot], sem)

        class _Pair:
            def __init__(self, a, b):
                self.a, self.b = a, b
            def start(self):
                self.a.start(); self.b.start()
            def wait(self):
                self.a.wait(); self.b.wait()

        def gather(idx_row, buf, sem):
            return pltpu.make_async_copy(
                nodes2_hbm.at[idx_row.at[pl.ds(0, 64)]], rows_v.at[buf], sem)

        def scat_add(buf, idx_row):
            pass

        idx_copy(src_hbm, src_g, 0, 0, semi).start()
        idx_copy(dst_hbm, dst_g, 0, 0, semi).start()
        pltpu.sync_copy(zeros_hbm,
                        acc_sh.at[pl.ds(sub * rows_per_tile, rows_per_tile)])
        idx_copy(src_hbm, src_g, 0, 0, semi).wait()
        idx_copy(dst_hbm, dst_g, 0, 0, semi).wait()
        plsc.subcore_barrier()

        gather(src_g.at[0].at[0], 0, sem0).start()
        for g in range(n_groups):
            cur, nxt = g % 3, (g + 1) % 3
            if g + 1 < n_groups:
                idx_copy(src_hbm, src_g, g + 1, nxt, semi).start()
                idx_copy(dst_hbm, dst_g, g + 1, nxt, semi).start()
            sg, dg = src_g.at[cur], dst_g.at[cur]

            @pl.loop(0, _G - 2, step=2)
            def _(c):
                gather(sg.at[c + 1], 1, sem1).start()
                gather(sg.at[c], 0, sem0).wait()
                scat_add(0, dg.at[c])
                gather(sg.at[c + 2], 0, sem0).start()
                gather(sg.at[c + 1], 1, sem1).wait()
                scat_add(1, dg.at[c + 1])

            gather(sg.at[_G - 1], 1, sem1).start()
            gather(sg.at[_G - 2], 0, sem0).wait()
            scat_add(0, dg.at[_G - 2])
            if g + 1 < n_groups:
                idx_copy(src_hbm, src_g, g + 1, nxt, semi).wait()
                idx_copy(dst_hbm, dst_g, g + 1, nxt, semi).wait()
                gather(src_g.at[nxt].at[0], 0, sem0).start()
            gather(sg.at[_G - 1], 1, sem1).wait()
            scat_add(1, dg.at[_G - 1])

        plsc.subcore_barrier()
        pltpu.sync_copy(
            acc_sh.at[pl.ds(sub * rows_per_tile, rows_per_tile)],
            out_hbm.at[core].at[pl.ds(sub * rows_per_tile, rows_per_tile)])

    return sck(nodes_ft, nodes_ft.reshape(n // 2, 2 * d) * 1.0, src_kc,
               dst_kc, zeros_rows)


def _gates(partials, h_in, bias, w6, b3, n, d):
    blk = 1000

    def body(part_ref, h_ref, bias_ref, w6_ref, b3_ref, out_ref):
        agg = part_ref[0] + part_ref[1] + bias_ref[...]
        m = jnp.max(agg, axis=-1, keepdims=True)
        e = jnp.exp(agg - m)
        a = e / jnp.sum(e, axis=-1, keepdims=True)
        h = h_ref[...]

        def mm(x, w):
            return jnp.dot(x, w, preferred_element_type=jnp.float32,
                           precision=lax.Precision.HIGHEST)

        r = jax.nn.sigmoid(mm(a, w6_ref[0]) + mm(h, w6_ref[1]) + b3_ref[0])
        z = jax.nn.sigmoid(mm(a, w6_ref[2]) + mm(h, w6_ref[3]) + b3_ref[1])
        hh = jnp.tanh(mm(a, w6_ref[4]) + mm(r * h, w6_ref[5]) + b3_ref[2])
        out_ref[...] = (1.0 - z) * h + z * hh

    return pl.pallas_call(
        body,
        grid=(n // blk,),
        in_specs=[
            pl.BlockSpec((2, blk, d), lambda i: (0, i, 0)),
            pl.BlockSpec((blk, d), lambda i: (i, 0)),
            pl.BlockSpec((1, d), lambda i: (0, 0)),
            pl.BlockSpec((6, d, d), lambda i: (0, 0, 0)),
            pl.BlockSpec((3, d), lambda i: (0, 0)),
        ],
        out_specs=pl.BlockSpec((blk, d), lambda i: (i, 0)),
        out_shape=jax.ShapeDtypeStruct((n, d), jnp.float32),
    )(partials, h_in, bias, w6, b3)


def kernel(nodes_ft, adj_list, bias, Wr, br, Wz, bz, Wt, bt):
    n, d = nodes_ft.shape
    e = adj_list.shape[1]
    if n % _NS:
        raise ValueError("N must divide the subcore count")
    rows_per_tile = -(-(n // _NS + 1) // 8) * 8
    n_pad = _NS * rows_per_tile
    k_chunks = -(-e // (_NW * _C * _G)) * _G
    e_pad = _NW * k_chunks * _C - e

    dst = adj_list[0]
    src = adj_list[1]
    src_p = jnp.concatenate([src, jnp.zeros((e_pad,), jnp.int32)])
    dst_pad = n + (jnp.arange(e_pad, dtype=jnp.int32) % (n_pad - n))
    dst_p = jnp.concatenate([dst, dst_pad])
    src_kc = (src_p // 2).reshape(_NW, k_chunks, _C)
    dst_kc = dst_p.reshape(_NW, k_chunks, _C)
    zeros_rows = jnp.zeros((rows_per_tile, d), jnp.float32)

    partials = _edge_aggregate(nodes_ft, src_kc, dst_kc, zeros_rows,
                               n, d, k_chunks, rows_per_tile)

    w6 = jnp.stack([Wr[:, :d].T, Wr[:, d:].T,
                    Wz[:, :d].T, Wz[:, d:].T,
                    Wt[:, :d].T, Wt[:, d:].T])
    b3 = jnp.stack([br, bz, bt])
    return _gates(partials, nodes_ft, bias, w6, b3, n, d)


# speedup vs baseline: 1.3586x; 1.3586x over previous
import dataclasses
import functools

import jax
import jax.numpy as jnp
from jax import lax
from jax.experimental import pallas as pl
from jax.experimental.pallas import tpu as pltpu
from jax.experimental.pallas import tpu_sc as plsc

_NC = 2
_NS = 16
_NW = _NC * _NS
_C = 128
_CAPC = 28
_CAP = _CAPC * _C


def _sc_compiler_params():
    cp = pltpu.CompilerParams()
    if "needs_layout_passes" in pltpu.CompilerParams.__dataclass_fields__:
        cp = dataclasses.replace(cp, needs_layout_passes=False)
    return cp


def _partition_edges(src_wk, dst_wk, half):
    kc = src_wk.shape[1]
    mesh = plsc.VectorSubcoreMesh(core_axis_name="c", subcore_axis_name="s")

    @functools.partial(
        pl.kernel,
        compiler_params=_sc_compiler_params(),
        out_type=(
            jax.ShapeDtypeStruct((_NW, 4, _CAP), jnp.int32),
            jax.ShapeDtypeStruct((_NW, 4, _CAP), jnp.int32),
            jax.ShapeDtypeStruct((_NW, 16), jnp.int32),
        ),
        mesh=mesh,
        scratch_types=[
            pltpu.VMEM((kc, _C), jnp.int32),
            pltpu.VMEM((kc, _C), jnp.int32),
            pltpu.VMEM((_CAP + 16,), jnp.int32),
            pltpu.VMEM((_CAP + 16,), jnp.int32),
            pltpu.VMEM((_CAP + 16,), jnp.int32),
            pltpu.VMEM((_CAP + 16,), jnp.int32),
            pltpu.VMEM((_CAP + 16,), jnp.int32),
            pltpu.VMEM((_CAP + 16,), jnp.int32),
            pltpu.VMEM((_CAP + 16,), jnp.int32),
            pltpu.VMEM((_CAP + 16,), jnp.int32),
            pltpu.VMEM((16,), jnp.int32),
        ],
    )
    def pk(src_hbm, dst_hbm, bsrc_hbm, bdst_hbm, cnt_hbm,
           sin, din, ss0, ss1, ss2, ss3, sd0, sd1, sd2, sd3, cnt_s):
        ssrc = (ss0, ss1, ss2, ss3)
        sdst = (sd0, sd1, sd2, sd3)
        core = lax.axis_index("c")
        sub = lax.axis_index("s")
        wid = core * _NS + sub
        pltpu.sync_copy(src_hbm.at[wid], sin)
        pltpu.sync_copy(dst_hbm.at[wid], din)

        fill_d = half + (lax.iota(jnp.int32, 16) & 7)
        fill_s = jnp.zeros((16,), jnp.int32)

        @pl.loop(0, _CAP + 16, step=16)
        def _(k):
            for b in range(4):
                ssrc[b][pl.ds(k, 16)] = fill_s
                sdst[b][pl.ds(k, 16)] = fill_d

        hvec = jnp.full((16,), half, jnp.int32)
        trash = jnp.full((16,), _CAP, jnp.int32)

        def body(k, ptrs):
            r = k // 8
            cl = (k % 8) * 16
            s16 = sin[r, pl.ds(cl, 16)]
            d16 = din[r, pl.ds(cl, 16)]
            ms = s16 < hvec
            md = d16 < hvec
            sr = jnp.where(ms, s16, s16 - hvec)
            dr = jnp.where(md, d16, d16 - hvec)
            masks = (ms & md, ms & (~md), (~ms) & md, (~ms) & (~md))
            out = []
            for b in range(4):
                m = masks[b]
                mi = m.astype(jnp.int32)
                pos = ptrs[b] + plsc.cumsum(mi) - mi
                idx = jnp.where(m, pos, trash)
                plsc.store_scatter(ssrc[b], [idx], sr)
                plsc.store_scatter(sdst[b], [idx], dr)
                out.append(ptrs[b] + plsc.all_reduce_population_count(m))
            return tuple(out)

        z = jnp.zeros((16,), jnp.int32)
        ptrs = lax.fori_loop(0, kc * 8, body, (z, z, z, z))

        lane = lax.iota(jnp.int32, 16)
        cvec = jnp.zeros((16,), jnp.int32)
        for b in range(4):
            cb = ((ptrs[b] + 255) // 256) * 2
            cvec = jnp.where(lane == b, cb, cvec)
        cnt_s[...] = cvec
        for b in range(4):
            pltpu.sync_copy(ssrc[b].at[pl.ds(0, _CAP)],
                            bsrc_hbm.at[wid].at[b])
            pltpu.sync_copy(sdst[b].at[pl.ds(0, _CAP)],
                            bdst_hbm.at[wid].at[b])
        pltpu.sync_copy(cnt_s, cnt_hbm.at[wid])

    return pk(src_wk, dst_wk)


def _aggregate(nodes_pad, bsrc4, bdst4, cnts, zeros_rows, half, hrows, d):
    stripe = hrows // _NS
    mesh = plsc.VectorSubcoreMesh(core_axis_name="c", subcore_axis_name="s")

    @functools.partial(
        pl.kernel,
        compiler_params=_sc_compiler_params(),
        out_type=jax.ShapeDtypeStruct((_NC, hrows, d), jnp.float32),
        mesh=mesh,
        scratch_types=[
            pltpu.VMEM((_CAPC, _C), jnp.int32),
            pltpu.VMEM((_CAPC, _C), jnp.int32),
            pltpu.VMEM((2, _C, d), jnp.float32),
            pltpu.VMEM_SHARED((hrows, d), jnp.float32),
            pltpu.VMEM_SHARED((hrows, d), jnp.float32),
            pltpu.VMEM((2, 16), jnp.int32),
            pltpu.SemaphoreType.DMA,
            pltpu.SemaphoreType.DMA,
        ],
    )
    def ak(nodes_hbm, bsrc_hbm, bdst_hbm, cnt_hbm, zeros_hbm, agg_hbm,
           idx_s, idx_d, rows_v, table_sh, acc_sh, cnt_v, sem0, sem1):
        core = lax.axis_index("c")
        sub = lax.axis_index("s")
        pltpu.sync_copy(cnt_hbm.at[sub], cnt_v.at[0])
        pltpu.sync_copy(cnt_hbm.at[sub + _NS], cnt_v.at[1])
        lane16 = lax.iota(jnp.int32, 16)
        pltpu.sync_copy(zeros_hbm, acc_sh.at[pl.ds(sub * stripe, stripe)])

        def gather(row, buf, sem):
            return pltpu.make_async_copy(
                table_sh.at[idx_s.at[row]], rows_v.at[buf], sem)

        for p in range(2):
            plsc.subcore_barrier()
            pltpu.sync_copy(
                nodes_hbm.at[pl.ds(p * half + sub * stripe, stripe)],
                table_sh.at[pl.ds(sub * stripe, stripe)])
            plsc.subcore_barrier()
            for li in range(2):
                bidx = 2 * p + core
                nc = jnp.sum(jnp.where(lane16 == bidx, cnt_v[li], 0))
                w = sub + _NS * li
                pltpu.sync_copy(bsrc_hbm.at[w].at[bidx], idx_s)
                pltpu.sync_copy(bdst_hbm.at[w].at[bidx], idx_d)
                gather(0, 0, sem0).start()

                def pair(i, carry):
                    c0 = 2 * i
                    gather(c0 + 1, 1, sem1).start()
                    gather(c0, 0, sem0).wait()
                    pltpu.sync_copy(rows_v.at[0], acc_sh.at[idx_d.at[c0]],
                                    add=True)
                    gather(c0 + 2, 0, sem0).start()
                    gather(c0 + 1, 1, sem1).wait()
                    pltpu.sync_copy(rows_v.at[1],
                                    acc_sh.at[idx_d.at[c0 + 1]], add=True)
                    return carry

                lax.fori_loop(0, nc // 2, pair, 0)
                gather(0, 0, sem0).wait()

        plsc.subcore_barrier()
        pltpu.sync_copy(acc_sh.at[pl.ds(sub * stripe, stripe)],
                        agg_hbm.at[core].at[pl.ds(sub * stripe, stripe)])

    return ak(nodes_pad, bsrc4, bdst4, cnts, zeros_rows)


def _gates(agg_pad, h_in, bias, w6, b3, n, d):
    blk = 1000
    nb = n // (2 * blk)

    def body(agg_ref, h_ref, bias_ref, w6_ref, b3_ref, out_ref):
        agg = agg_ref[0] + bias_ref[...]
        m = jnp.max(agg, axis=-1, keepdims=True)
        e = jnp.exp(agg - m)
        a = e / jnp.sum(e, axis=-1, keepdims=True)
        h = h_ref[...]

        def mm(x, w):
            return jnp.dot(x, w, preferred_element_type=jnp.float32,
                           precision=lax.Precision.HIGHEST)

        r = jax.nn.sigmoid(mm(a, w6_ref[0]) + mm(h, w6_ref[1]) + b3_ref[0])
        z = jax.nn.sigmoid(mm(a, w6_ref[2]) + mm(h, w6_ref[3]) + b3_ref[1])
        hh = jnp.tanh(mm(a, w6_ref[4]) + mm(r * h, w6_ref[5]) + b3_ref[2])
        out_ref[...] = (1.0 - z) * h + z * hh

    return pl.pallas_call(
        body,
        grid=(n // blk,),
        in_specs=[
            pl.BlockSpec((1, blk, d), lambda i, nb=nb: (i // nb, i % nb, 0)),
            pl.BlockSpec((blk, d), lambda i: (i, 0)),
            pl.BlockSpec((1, d), lambda i: (0, 0)),
            pl.BlockSpec((6, d, d), lambda i: (0, 0, 0)),
            pl.BlockSpec((3, d), lambda i: (0, 0)),
        ],
        out_specs=pl.BlockSpec((blk, d), lambda i: (i, 0)),
        out_shape=jax.ShapeDtypeStruct((n, d), jnp.float32),
    )(agg_pad, h_in, bias, w6, b3)


def kernel(nodes_ft, adj_list, bias, Wr, br, Wz, bz, Wt, bt):
    n, d = nodes_ft.shape
    e = adj_list.shape[1]
    half = n // 2
    hrows = _NS * (-(-(half + _C // 2) // (8 * _NS)) * 8)
    n_dummy = hrows - half

    epw = -(-e // _NW)
    epw_pad = -(-epw // _C) * _C
    pw = epw_pad - epw
    dst = adj_list[0].reshape(_NW, epw)
    src = adj_list[1].reshape(_NW, epw)
    pad_d = jnp.broadcast_to(n + (jnp.arange(pw, dtype=jnp.int32) % n_dummy),
                             (_NW, pw))
    pad_s = jnp.zeros((_NW, pw), jnp.int32)
    src_wk = jnp.concatenate([src, pad_s], axis=1).reshape(_NW, -1, _C)
    dst_wk = jnp.concatenate([dst, pad_d], axis=1).reshape(_NW, -1, _C)

    bsrc, bdst, cnts = _partition_edges(src_wk, dst_wk, half)
    bsrc4 = bsrc.reshape(_NW, 4, _CAPC, _C)
    bdst4 = bdst.reshape(_NW, 4, _CAPC, _C)

    nodes_pad = jnp.pad(nodes_ft, ((0, 2 * hrows - n), (0, 0)))
    zeros_rows = jnp.zeros((hrows // _NS, d), jnp.float32)
    agg_pad = _aggregate(nodes_pad, bsrc4, bdst4, cnts, zeros_rows,
                         half, hrows, d)

    w6 = jnp.stack([Wr[:, :d].T, Wr[:, d:].T,
                    Wz[:, :d].T, Wz[:, d:].T,
                    Wt[:, :d].T, Wt[:, d:].T])
    b3 = jnp.stack([br, bz, bt])
    return _gates(agg_pad, nodes_ft, bias, w6, b3, n, d)


# speedup vs baseline: 1.3837x; 1.0185x over previous
import dataclasses
import functools

import jax
import jax.numpy as jnp
from jax import lax
from jax.experimental import pallas as pl
from jax.experimental.pallas import tpu as pltpu
from jax.experimental.pallas import tpu_sc as plsc

_NC = 2
_NS = 16
_NW = _NC * _NS
_C = 128
_CAPC = 28
_CAP = _CAPC * _C


def _sc_compiler_params():
    cp = pltpu.CompilerParams()
    if "needs_layout_passes" in pltpu.CompilerParams.__dataclass_fields__:
        cp = dataclasses.replace(cp, needs_layout_passes=False)
    return cp


def _partition_edges(src_wk, dst_wk, half):
    kc = src_wk.shape[1]
    mesh = plsc.VectorSubcoreMesh(core_axis_name="c", subcore_axis_name="s")

    @functools.partial(
        pl.kernel,
        compiler_params=_sc_compiler_params(),
        out_type=(
            jax.ShapeDtypeStruct((_NW, 4, _CAP), jnp.int32),
            jax.ShapeDtypeStruct((_NW, 4, _CAP), jnp.int32),
            jax.ShapeDtypeStruct((_NW, 16), jnp.int32),
        ),
        mesh=mesh,
        scratch_types=[
            pltpu.VMEM((kc, _C), jnp.int32),
            pltpu.VMEM((kc, _C), jnp.int32),
            pltpu.VMEM((_CAP + 16,), jnp.int32),
            pltpu.VMEM((_CAP + 16,), jnp.int32),
            pltpu.VMEM((_CAP + 16,), jnp.int32),
            pltpu.VMEM((_CAP + 16,), jnp.int32),
            pltpu.VMEM((_CAP + 16,), jnp.int32),
            pltpu.VMEM((_CAP + 16,), jnp.int32),
            pltpu.VMEM((_CAP + 16,), jnp.int32),
            pltpu.VMEM((_CAP + 16,), jnp.int32),
            pltpu.VMEM((16,), jnp.int32),
        ],
    )
    def pk(src_hbm, dst_hbm, bsrc_hbm, bdst_hbm, cnt_hbm,
           sin, din, ss0, ss1, ss2, ss3, sd0, sd1, sd2, sd3, cnt_s):
        ssrc = (ss0, ss1, ss2, ss3)
        sdst = (sd0, sd1, sd2, sd3)
        core = lax.axis_index("c")
        sub = lax.axis_index("s")
        wid = core * _NS + sub
        pltpu.sync_copy(src_hbm.at[wid], sin)
        pltpu.sync_copy(dst_hbm.at[wid], din)

        fill_d = half + (lax.iota(jnp.int32, 16) & 7)
        fill_s = jnp.zeros((16,), jnp.int32)

        @pl.loop(0, _CAP + 16, step=16)
        def _(k):
            for b in range(4):
                ssrc[b][pl.ds(k, 16)] = fill_s
                sdst[b][pl.ds(k, 16)] = fill_d

        hvec = jnp.full((16,), half, jnp.int32)
        trash = jnp.full((16,), _CAP, jnp.int32)

        def body(k, ptrs):
            r = k // 8
            cl = (k % 8) * 16
            s16 = sin[r, pl.ds(cl, 16)]
            d16 = din[r, pl.ds(cl, 16)]
            ms = s16 < hvec
            md = d16 < hvec
            sr = jnp.where(ms, s16, s16 - hvec)
            dr = jnp.where(md, d16, d16 - hvec)
            masks = (ms & md, ms & (~md), (~ms) & md, (~ms) & (~md))
            out = []
            for b in range(4):
                m = masks[b]
                mi = m.astype(jnp.int32)
                pos = ptrs[b] + plsc.cumsum(mi) - mi
                idx = jnp.where(m, pos, trash)
                plsc.store_scatter(ssrc[b], [idx], sr)
                plsc.store_scatter(sdst[b], [idx], dr)
                out.append(ptrs[b] + plsc.all_reduce_population_count(m))
            return tuple(out)

        z = jnp.zeros((16,), jnp.int32)
        ptrs = lax.fori_loop(0, kc * 8, body, (z, z, z, z))

        lane = lax.iota(jnp.int32, 16)
        cvec = jnp.zeros((16,), jnp.int32)
        for b in range(4):
            cb = ((ptrs[b] + 255) // 256) * 2
            cvec = jnp.where(lane == b, cb, cvec)
        cnt_s[...] = cvec
        for b in range(4):
            pltpu.sync_copy(ssrc[b].at[pl.ds(0, _CAP)],
                            bsrc_hbm.at[wid].at[b])
            pltpu.sync_copy(sdst[b].at[pl.ds(0, _CAP)],
                            bdst_hbm.at[wid].at[b])
        pltpu.sync_copy(cnt_s, cnt_hbm.at[wid])

    return pk(src_wk, dst_wk)


def _aggregate(nodes_pad, bsrc4, bdst4, cnts, zeros_rows, half, hrows, d):
    stripe = hrows // _NS
    mesh = plsc.VectorSubcoreMesh(core_axis_name="c", subcore_axis_name="s")

    @functools.partial(
        pl.kernel,
        compiler_params=_sc_compiler_params(),
        out_type=jax.ShapeDtypeStruct((_NC, hrows, d), jnp.float32),
        mesh=mesh,
        scratch_types=[
            pltpu.VMEM((_CAPC, _C), jnp.int32),
            pltpu.VMEM((_CAPC, _C), jnp.int32),
            pltpu.VMEM((2, _C, d), jnp.float32),
            pltpu.VMEM_SHARED((hrows, d), jnp.float32),
            pltpu.VMEM_SHARED((hrows, d), jnp.float32),
            pltpu.VMEM((2, 16), jnp.int32),
            pltpu.SemaphoreType.DMA,
            pltpu.SemaphoreType.DMA,
            pltpu.SemaphoreType.DMA,
            pltpu.SemaphoreType.DMA,
        ],
    )
    def ak(nodes_hbm, bsrc_hbm, bdst_hbm, cnt_hbm, zeros_hbm, agg_hbm,
           idx_s, idx_d, rows_v, table_sh, acc_sh, cnt_v,
           sem0, sem1, sems0, sems1):
        core = lax.axis_index("c")
        sub = lax.axis_index("s")
        pltpu.sync_copy(cnt_hbm.at[sub], cnt_v.at[0])
        pltpu.sync_copy(cnt_hbm.at[sub + _NS], cnt_v.at[1])
        lane16 = lax.iota(jnp.int32, 16)
        pltpu.sync_copy(zeros_hbm, acc_sh.at[pl.ds(sub * stripe, stripe)])

        def gather(row, buf, sem):
            return pltpu.make_async_copy(
                table_sh.at[idx_s.at[row]], rows_v.at[buf], sem)

        def scat(row, buf, sem):
            return pltpu.make_async_copy(
                rows_v.at[buf], acc_sh.at[idx_d.at[row]], sem)

        for p in range(2):
            plsc.subcore_barrier()
            pltpu.sync_copy(
                nodes_hbm.at[pl.ds(p * half + sub * stripe, stripe)],
                table_sh.at[pl.ds(sub * stripe, stripe)])
            plsc.subcore_barrier()
            for li in range(2):
                bidx = 2 * p + core
                nc = jnp.sum(jnp.where(lane16 == bidx, cnt_v[li], 0))
                w = sub + _NS * li
                pltpu.sync_copy(bsrc_hbm.at[w].at[bidx], idx_s)
                pltpu.sync_copy(bdst_hbm.at[w].at[bidx], idx_d)
                gather(0, 0, sem0).start()
                gather(1, 1, sem1).start()

                def pair(i, carry):
                    c0 = 2 * i
                    gather(c0, 0, sem0).wait()
                    scat(c0, 0, sems0).start(add=True)
                    gather(c0 + 1, 1, sem1).wait()
                    scat(c0 + 1, 1, sems1).start(add=True)
                    scat(c0, 0, sems0).wait()
                    gather(c0 + 2, 0, sem0).start()
                    scat(c0 + 1, 1, sems1).wait()
                    gather(c0 + 3, 1, sem1).start()
                    return carry

                lax.fori_loop(0, nc // 2, pair, 0)
                gather(0, 0, sem0).wait()
                gather(1, 1, sem1).wait()

        plsc.subcore_barrier()
        pltpu.sync_copy(acc_sh.at[pl.ds(sub * stripe, stripe)],
                        agg_hbm.at[core].at[pl.ds(sub * stripe, stripe)])

    return ak(nodes_pad, bsrc4, bdst4, cnts, zeros_rows)


def _gates(agg_pad, h_in, bias, w6, b3, n, d):
    blk = 1000
    nb = n // (2 * blk)

    def body(agg_ref, h_ref, bias_ref, w6_ref, b3_ref, out_ref):
        agg = agg_ref[0] + bias_ref[...]
        m = jnp.max(agg, axis=-1, keepdims=True)
        e = jnp.exp(agg - m)
        a = e / jnp.sum(e, axis=-1, keepdims=True)
        h = h_ref[...]

        def mm(x, w):
            return jnp.dot(x, w, preferred_element_type=jnp.float32)

        r = jax.nn.sigmoid(mm(a, w6_ref[0]) + mm(h, w6_ref[1]) + b3_ref[0])
        z = jax.nn.sigmoid(mm(a, w6_ref[2]) + mm(h, w6_ref[3]) + b3_ref[1])
        hh = jnp.tanh(mm(a, w6_ref[4]) + mm(r * h, w6_ref[5]) + b3_ref[2])
        out_ref[...] = (1.0 - z) * h + z * hh

    return pl.pallas_call(
        body,
        grid=(n // blk,),
        in_specs=[
            pl.BlockSpec((1, blk, d), lambda i, nb=nb: (i // nb, i % nb, 0)),
            pl.BlockSpec((blk, d), lambda i: (i, 0)),
            pl.BlockSpec((1, d), lambda i: (0, 0)),
            pl.BlockSpec((6, d, d), lambda i: (0, 0, 0)),
            pl.BlockSpec((3, d), lambda i: (0, 0)),
        ],
        out_specs=pl.BlockSpec((blk, d), lambda i: (i, 0)),
        out_shape=jax.ShapeDtypeStruct((n, d), jnp.float32),
    )(agg_pad, h_in, bias, w6, b3)


def kernel(nodes_ft, adj_list, bias, Wr, br, Wz, bz, Wt, bt):
    n, d = nodes_ft.shape
    e = adj_list.shape[1]
    half = n // 2
    hrows = _NS * (-(-(half + _C // 2) // (8 * _NS)) * 8)
    n_dummy = hrows - half

    epw = -(-e // _NW)
    epw_pad = -(-epw // _C) * _C
    pw = epw_pad - epw
    dst = adj_list[0].reshape(_NW, epw)
    src = adj_list[1].reshape(_NW, epw)
    pad_d = jnp.broadcast_to(n + (jnp.arange(pw, dtype=jnp.int32) % n_dummy),
                             (_NW, pw))
    pad_s = jnp.zeros((_NW, pw), jnp.int32)
    src_wk = jnp.concatenate([src, pad_s], axis=1).reshape(_NW, -1, _C)
    dst_wk = jnp.concatenate([dst, pad_d], axis=1).reshape(_NW, -1, _C)

    bsrc, bdst, cnts = _partition_edges(src_wk, dst_wk, half)
    bsrc4 = bsrc.reshape(_NW, 4, _CAPC, _C)
    bdst4 = bdst.reshape(_NW, 4, _CAPC, _C)

    nodes_pad = jnp.pad(nodes_ft, ((0, 2 * hrows - n), (0, 0)))
    zeros_rows = jnp.zeros((hrows // _NS, d), jnp.float32)
    agg_pad = _aggregate(nodes_pad, bsrc4, bdst4, cnts, zeros_rows,
                         half, hrows, d)

    w6 = jnp.stack([Wr[:, :d].T, Wr[:, d:].T,
                    Wz[:, :d].T, Wz[:, d:].T,
                    Wt[:, :d].T, Wt[:, d:].T])
    b3 = jnp.stack([br, bz, bt])
    return _gates(agg_pad, nodes_ft, bias, w6, b3, n, d)


# speedup vs baseline: 1.5623x; 1.1290x over previous
import dataclasses
import functools

import jax
import jax.numpy as jnp
from jax import lax
from jax.experimental import pallas as pl
from jax.experimental.pallas import tpu as pltpu
from jax.experimental.pallas import tpu_sc as plsc

_NC = 2
_NS = 16
_NW = _NC * _NS
_C = 128
_CAPC = 28
_CAP = _CAPC * _C


def _sc_compiler_params():
    cp = pltpu.CompilerParams()
    if "needs_layout_passes" in pltpu.CompilerParams.__dataclass_fields__:
        cp = dataclasses.replace(cp, needs_layout_passes=False)
    return cp


def _partition_edges(src_wk, dst_wk, half):
    kc = src_wk.shape[1]
    mesh = plsc.VectorSubcoreMesh(core_axis_name="c", subcore_axis_name="s")

    @functools.partial(
        pl.kernel,
        compiler_params=_sc_compiler_params(),
        out_type=(
            jax.ShapeDtypeStruct((_NW, 4, _CAP), jnp.int32),
            jax.ShapeDtypeStruct((_NW, 4, _CAP), jnp.int32),
            jax.ShapeDtypeStruct((_NW, 16), jnp.int32),
        ),
        mesh=mesh,
        scratch_types=[
            pltpu.VMEM((kc, _C), jnp.int32),
            pltpu.VMEM((kc, _C), jnp.int32),
            pltpu.VMEM((_CAP + 16,), jnp.int32),
            pltpu.VMEM((_CAP + 16,), jnp.int32),
            pltpu.VMEM((_CAP + 16,), jnp.int32),
            pltpu.VMEM((_CAP + 16,), jnp.int32),
            pltpu.VMEM((_CAP + 16,), jnp.int32),
            pltpu.VMEM((_CAP + 16,), jnp.int32),
            pltpu.VMEM((_CAP + 16,), jnp.int32),
            pltpu.VMEM((_CAP + 16,), jnp.int32),
            pltpu.VMEM((16,), jnp.int32),
        ],
    )
    def pk(src_hbm, dst_hbm, bsrc_hbm, bdst_hbm, cnt_hbm,
           sin, din, ss0, ss1, ss2, ss3, sd0, sd1, sd2, sd3, cnt_s):
        ssrc = (ss0, ss1, ss2, ss3)
        sdst = (sd0, sd1, sd2, sd3)
        core = lax.axis_index("c")
        sub = lax.axis_index("s")
        wid = core * _NS + sub
        pltpu.sync_copy(src_hbm.at[wid], sin)
        pltpu.sync_copy(dst_hbm.at[wid], din)

        fill_d = half + (lax.iota(jnp.int32, 16) & 7)
        fill_s = jnp.zeros((16,), jnp.int32)

        @pl.loop(0, _CAP + 16, step=16)
        def _(k):
            for b in range(4):
                ssrc[b][pl.ds(k, 16)] = fill_s
                sdst[b][pl.ds(k, 16)] = fill_d

        hvec = jnp.full((16,), half, jnp.int32)
        trash = jnp.full((16,), _CAP, jnp.int32)

        def body(k, ptrs):
            r = k // 8
            cl = (k % 8) * 16
            s16 = sin[r, pl.ds(cl, 16)]
            d16 = din[r, pl.ds(cl, 16)]
            ms = s16 < hvec
            md = d16 < hvec
            sr = jnp.where(ms, s16, s16 - hvec)
            dr = jnp.where(md, d16, d16 - hvec)
            masks = (ms & md, ms & (~md), (~ms) & md, (~ms) & (~md))
            out = []
            for b in range(4):
                m = masks[b]
                mi = m.astype(jnp.int32)
                pos = ptrs[b] + plsc.cumsum(mi) - mi
                idx = jnp.where(m, pos, trash)
                plsc.store_scatter(ssrc[b], [idx], sr)
                plsc.store_scatter(sdst[b], [idx], dr)
                out.append(ptrs[b] + plsc.all_reduce_population_count(m))
            return tuple(out)

        z = jnp.zeros((16,), jnp.int32)
        ptrs = lax.fori_loop(0, kc * 8, body, (z, z, z, z))

        lane = lax.iota(jnp.int32, 16)
        cvec = jnp.zeros((16,), jnp.int32)
        for b in range(4):
            cb = ((ptrs[b] + 255) // 256) * 2
            cvec = jnp.where(lane == b, cb, cvec)
        cnt_s[...] = cvec
        for b in range(4):
            pltpu.sync_copy(ssrc[b].at[pl.ds(0, _CAP)],
                            bsrc_hbm.at[wid].at[b])
            pltpu.sync_copy(sdst[b].at[pl.ds(0, _CAP)],
                            bdst_hbm.at[wid].at[b])
        pltpu.sync_copy(cnt_s, cnt_hbm.at[wid])

    return pk(src_wk, dst_wk)


def _aggregate(nodes_pad, bsrc4, bdst4, cnts, zeros_rows, half, hrows, d):
    stripe = hrows // _NS
    mesh = plsc.VectorSubcoreMesh(core_axis_name="c", subcore_axis_name="s")

    @functools.partial(
        pl.kernel,
        compiler_params=_sc_compiler_params(),
        out_type=jax.ShapeDtypeStruct((_NC, hrows, d), jnp.float32),
        mesh=mesh,
        scratch_types=[
            pltpu.VMEM((_CAPC, _C), jnp.int32),
            pltpu.VMEM((_CAPC, _C), jnp.int32),
            pltpu.VMEM((2, _C, d), jnp.float32),
            pltpu.VMEM_SHARED((hrows, d), jnp.float32),
            pltpu.VMEM_SHARED((hrows, d), jnp.float32),
            pltpu.VMEM((2, 16), jnp.int32),
            pltpu.SemaphoreType.DMA,
            pltpu.SemaphoreType.DMA,
            pltpu.SemaphoreType.DMA,
            pltpu.SemaphoreType.DMA,
        ],
    )
    def ak(nodes_hbm, bsrc_hbm, bdst_hbm, cnt_hbm, zeros_hbm, agg_hbm,
           idx_s, idx_d, rows_v, table_sh, acc_sh, cnt_v,
           sem0, sem1, sems0, sems1):
        core = lax.axis_index("c")
        sub = lax.axis_index("s")
        pltpu.sync_copy(cnt_hbm.at[sub], cnt_v.at[0])
        pltpu.sync_copy(cnt_hbm.at[sub + _NS], cnt_v.at[1])
        lane16 = lax.iota(jnp.int32, 16)
        pltpu.sync_copy(zeros_hbm, acc_sh.at[pl.ds(sub * stripe, stripe)])

        def gather(row, buf, sem):
            return pltpu.make_async_copy(
                table_sh.at[idx_s.at[row]], rows_v.at[buf], sem)

        def scat(row, buf, sem):
            return pltpu.make_async_copy(
                rows_v.at[buf], acc_sh.at[idx_d.at[row]], sem)

        for p in range(2):
            plsc.subcore_barrier()
            pltpu.sync_copy(
                nodes_hbm.at[pl.ds(p * half + sub * stripe, stripe)],
                table_sh.at[pl.ds(sub * stripe, stripe)])
            plsc.subcore_barrier()
            for li in range(2):
                bidx = 2 * p + core
                nc = jnp.sum(jnp.where(lane16 == bidx, cnt_v[li], 0))
                w = sub + _NS * li
                pltpu.sync_copy(bsrc_hbm.at[w].at[bidx], idx_s)
                pltpu.sync_copy(bdst_hbm.at[w].at[bidx], idx_d)
                gather(0, 0, sem0).start()

                def pair(i, carry):
                    c0 = 2 * i
                    gather(c0 + 1, 1, sem1).start()
                    gather(c0, 0, sem0).wait()
                    pltpu.sync_copy(rows_v.at[0], acc_sh.at[idx_d.at[c0]],
                                    add=True)
                    gather(c0 + 2, 0, sem0).start()
                    gather(c0 + 1, 1, sem1).wait()
                    pltpu.sync_copy(rows_v.at[1],
                                    acc_sh.at[idx_d.at[c0 + 1]], add=True)
                    return carry

                lax.fori_loop(0, nc // 2, pair, 0)
                gather(0, 0, sem0).wait()

        plsc.subcore_barrier()
        pltpu.sync_copy(acc_sh.at[pl.ds(sub * stripe, stripe)],
                        agg_hbm.at[core].at[pl.ds(sub * stripe, stripe)])

    return ak(nodes_pad, bsrc4, bdst4, cnts, zeros_rows)


def _gates(agg_pad, h_in, bias, w6, b3, n, d):
    blk = 1000
    nb = n // (2 * blk)

    def body(agg_ref, h_ref, bias_ref, w6_ref, b3_ref, out_ref):
        agg = agg_ref[0] + bias_ref[...]
        m = jnp.max(agg, axis=-1, keepdims=True)
        e = jnp.exp(agg - m)
        a = e / jnp.sum(e, axis=-1, keepdims=True)
        h = h_ref[...]

        def mm(x, w):
            return jnp.dot(x, w, preferred_element_type=jnp.float32)

        r = jax.nn.sigmoid(mm(a, w6_ref[0]) + mm(h, w6_ref[1]) + b3_ref[0])
        z = jax.nn.sigmoid(mm(a, w6_ref[2]) + mm(h, w6_ref[3]) + b3_ref[1])
        hh = jnp.tanh(mm(a, w6_ref[4]) + mm(r * h, w6_ref[5]) + b3_ref[2])
        out_ref[...] = (1.0 - z) * h + z * hh

    return pl.pallas_call(
        body,
        grid=(n // blk,),
        in_specs=[
            pl.BlockSpec((1, blk, d), lambda i, nb=nb: (i // nb, i % nb, 0)),
            pl.BlockSpec((blk, d), lambda i: (i, 0)),
            pl.BlockSpec((1, d), lambda i: (0, 0)),
            pl.BlockSpec((6, d, d), lambda i: (0, 0, 0)),
            pl.BlockSpec((3, d), lambda i: (0, 0)),
        ],
        out_specs=pl.BlockSpec((blk, d), lambda i: (i, 0)),
        out_shape=jax.ShapeDtypeStruct((n, d), jnp.float32),
    )(agg_pad, h_in, bias, w6, b3)


def kernel(nodes_ft, adj_list, bias, Wr, br, Wz, bz, Wt, bt):
    n, d = nodes_ft.shape
    e = adj_list.shape[1]
    half = n // 2
    hrows = _NS * (-(-(half + _C // 2) // (8 * _NS)) * 8)
    n_dummy = hrows - half

    epw = -(-e // _NW)
    epw_pad = -(-epw // _C) * _C
    pw = epw_pad - epw
    dst = adj_list[0].reshape(_NW, epw)
    src = adj_list[1].reshape(_NW, epw)
    pad_d = jnp.broadcast_to(n + (jnp.arange(pw, dtype=jnp.int32) % n_dummy),
                             (_NW, pw))
    pad_s = jnp.zeros((_NW, pw), jnp.int32)
    src_wk = jnp.concatenate([src, pad_s], axis=1).reshape(_NW, -1, _C)
    dst_wk = jnp.concatenate([dst, pad_d], axis=1).reshape(_NW, -1, _C)

    bsrc, bdst, cnts = _partition_edges(src_wk, dst_wk, half)
    bsrc4 = bsrc.reshape(_NW, 4, _CAPC, _C)
    bdst4 = bdst.reshape(_NW, 4, _CAPC, _C)

    nodes_pad = jnp.pad(nodes_ft, ((0, 2 * hrows - n), (0, 0)))
    zeros_rows = jnp.zeros((hrows // _NS, d), jnp.float32)
    agg_pad = _aggregate(nodes_pad, bsrc4, bdst4, cnts, zeros_rows,
                         half, hrows, d)

    w6 = jnp.stack([Wr[:, :d].T, Wr[:, d:].T,
                    Wz[:, :d].T, Wz[:, d:].T,
                    Wt[:, :d].T, Wt[:, d:].T])
    b3 = jnp.stack([br, bz, bt])
    return _gates(agg_pad, nodes_ft, bias, w6, b3, n, d)


# speedup vs baseline: 1.6111x; 1.0312x over previous
import dataclasses
import functools

import jax
import jax.numpy as jnp
from jax import lax
from jax.experimental import pallas as pl
from jax.experimental.pallas import tpu as pltpu
from jax.experimental.pallas import tpu_sc as plsc

_NC = 2
_NS = 16
_NW = _NC * _NS
_C = 128
_CAPC = 28
_CAP = _CAPC * _C


def _sc_compiler_params():
    cp = pltpu.CompilerParams()
    if "needs_layout_passes" in pltpu.CompilerParams.__dataclass_fields__:
        cp = dataclasses.replace(cp, needs_layout_passes=False)
    return cp


def _partition_edges(src_wk, dst_wk, half):
    kc = src_wk.shape[1]
    mesh = plsc.VectorSubcoreMesh(core_axis_name="c", subcore_axis_name="s")

    @functools.partial(
        pl.kernel,
        compiler_params=_sc_compiler_params(),
        out_type=(
            jax.ShapeDtypeStruct((_NW, 4, _CAP), jnp.int32),
            jax.ShapeDtypeStruct((_NW, 4, _CAP), jnp.int32),
            jax.ShapeDtypeStruct((_NW, 16), jnp.int32),
        ),
        mesh=mesh,
        scratch_types=[
            pltpu.VMEM((kc, _C), jnp.int32),
            pltpu.VMEM((kc, _C), jnp.int32),
            pltpu.VMEM((_CAP + 16,), jnp.int32),
            pltpu.VMEM((_CAP + 16,), jnp.int32),
            pltpu.VMEM((_CAP + 16,), jnp.int32),
            pltpu.VMEM((_CAP + 16,), jnp.int32),
            pltpu.VMEM((_CAP + 16,), jnp.int32),
            pltpu.VMEM((_CAP + 16,), jnp.int32),
            pltpu.VMEM((_CAP + 16,), jnp.int32),
            pltpu.VMEM((_CAP + 16,), jnp.int32),
            pltpu.VMEM((16,), jnp.int32),
        ],
    )
    def pk(src_hbm, dst_hbm, bsrc_hbm, bdst_hbm, cnt_hbm,
           sin, din, ss0, ss1, ss2, ss3, sd0, sd1, sd2, sd3, cnt_s):
        ssrc = (ss0, ss1, ss2, ss3)
        sdst = (sd0, sd1, sd2, sd3)
        core = lax.axis_index("c")
        sub = lax.axis_index("s")
        wid = core * _NS + sub
        pltpu.sync_copy(src_hbm.at[wid], sin)
        pltpu.sync_copy(dst_hbm.at[wid], din)

        fill_d = half + (lax.iota(jnp.int32, 16) & 7)
        fill_s = jnp.zeros((16,), jnp.int32)

        @pl.loop(0, _CAP + 16, step=16)
        def _(k):
            for b in range(4):
                ssrc[b][pl.ds(k, 16)] = fill_s
                sdst[b][pl.ds(k, 16)] = fill_d

        hvec = jnp.full((16,), half, jnp.int32)
        trash = jnp.full((16,), _CAP, jnp.int32)

        def body(k, ptrs):
            r = k // 8
            cl = (k % 8) * 16
            s16 = sin[r, pl.ds(cl, 16)]
            d16 = din[r, pl.ds(cl, 16)]
            ms = s16 < hvec
            md = d16 < hvec
            sr = jnp.where(ms, s16, s16 - hvec)
            dr = jnp.where(md, d16, d16 - hvec)
            masks = (ms & md, ms & (~md), (~ms) & md, (~ms) & (~md))
            out = []
            for b in range(4):
                m = masks[b]
                mi = m.astype(jnp.int32)
                pos = ptrs[b] + plsc.cumsum(mi) - mi
                idx = jnp.where(m, pos, trash)
                plsc.store_scatter(ssrc[b], [idx], sr)
                plsc.store_scatter(sdst[b], [idx], dr)
                out.append(ptrs[b] + plsc.all_reduce_population_count(m))
            return tuple(out)

        z = jnp.zeros((16,), jnp.int32)
        ptrs = lax.fori_loop(0, kc * 8, body, (z, z, z, z))

        lane = lax.iota(jnp.int32, 16)
        cvec = jnp.zeros((16,), jnp.int32)
        for b in range(4):
            cb = ((ptrs[b] + 255) // 256) * 2
            cvec = jnp.where(lane == b, cb, cvec)
        cnt_s[...] = cvec
        for b in range(4):
            pltpu.sync_copy(ssrc[b].at[pl.ds(0, _CAP)],
                            bsrc_hbm.at[wid].at[b])
            pltpu.sync_copy(sdst[b].at[pl.ds(0, _CAP)],
                            bdst_hbm.at[wid].at[b])
        pltpu.sync_copy(cnt_s, cnt_hbm.at[wid])

    return pk(src_wk, dst_wk)


def _aggregate(nodes_pad, bsrc4, bdst4, cnts, zeros_rows, half, hrows, d):
    stripe = hrows // _NS
    mesh = plsc.VectorSubcoreMesh(core_axis_name="c", subcore_axis_name="s")

    @functools.partial(
        pl.kernel,
        compiler_params=_sc_compiler_params(),
        out_type=jax.ShapeDtypeStruct((_NC, hrows, d), jnp.float32),
        mesh=mesh,
        scratch_types=[
            pltpu.VMEM((2 * _CAPC, _C), jnp.int32),
            pltpu.VMEM((2 * _CAPC, _C), jnp.int32),
            pltpu.VMEM((2, _C, d), jnp.float32),
            pltpu.VMEM_SHARED((hrows, d), jnp.float32),
            pltpu.VMEM_SHARED((hrows, d), jnp.float32),
            pltpu.VMEM((2, 16), jnp.int32),
            pltpu.SemaphoreType.DMA,
            pltpu.SemaphoreType.DMA,
            pltpu.SemaphoreType.DMA,
        ],
    )
    def ak(nodes_hbm, bsrc_hbm, bdst_hbm, cnt_hbm, zeros_hbm, agg_hbm,
           idx_s, idx_d, rows_v, table_sh, acc_sh, cnt_v,
           sem0, sem1, semi):
        core = lax.axis_index("c")
        sub = lax.axis_index("s")
        pltpu.sync_copy(cnt_hbm.at[sub], cnt_v.at[0])
        pltpu.sync_copy(cnt_hbm.at[sub + _NS], cnt_v.at[1])
        lane16 = lax.iota(jnp.int32, 16)
        pltpu.sync_copy(zeros_hbm, acc_sh.at[pl.ds(sub * stripe, stripe)])

        def idx_load(k, slot):
            p, li = k // 2, k % 2
            bidx = 2 * p + core
            w = sub + _NS * li
            sl = pl.ds(slot * _CAPC, _CAPC)
            return (pltpu.make_async_copy(bsrc_hbm.at[w].at[bidx],
                                          idx_s.at[sl], semi),
                    pltpu.make_async_copy(bdst_hbm.at[w].at[bidx],
                                          idx_d.at[sl], semi))

        for cp in idx_load(0, 0):
            cp.start()
        for cp in idx_load(0, 0):
            cp.wait()
        for k in range(4):
            p, li = k // 2, k % 2
            slot = k % 2
            if li == 0:
                plsc.subcore_barrier()
                pltpu.sync_copy(
                    nodes_hbm.at[pl.ds(p * half + sub * stripe, stripe)],
                    table_sh.at[pl.ds(sub * stripe, stripe)])
                plsc.subcore_barrier()
            if k + 1 < 4:
                for cp in idx_load(k + 1, (k + 1) % 2):
                    cp.start()

            def gather(row, buf, sem, base=slot * _CAPC):
                return pltpu.make_async_copy(
                    table_sh.at[idx_s.at[base + row]], rows_v.at[buf], sem)

            bidx = 2 * p + core
            nc = jnp.sum(jnp.where(lane16 == bidx, cnt_v[li], 0))
            gather(0, 0, sem0).start()

            def pair(i, carry, gather=gather, slot=slot):
                c0 = 2 * i
                gather(c0 + 1, 1, sem1).start()
                gather(c0, 0, sem0).wait()
                pltpu.sync_copy(rows_v.at[0],
                                acc_sh.at[idx_d.at[slot * _CAPC + c0]],
                                add=True)
                gather(c0 + 2, 0, sem0).start()
                gather(c0 + 1, 1, sem1).wait()
                pltpu.sync_copy(rows_v.at[1],
                                acc_sh.at[idx_d.at[slot * _CAPC + c0 + 1]],
                                add=True)
                return carry

            lax.fori_loop(0, nc // 2, pair, 0)
            gather(0, 0, sem0).wait()
            if k + 1 < 4:
                for cp in idx_load(k + 1, (k + 1) % 2):
                    cp.wait()

        plsc.subcore_barrier()
        pltpu.sync_copy(acc_sh.at[pl.ds(sub * stripe, stripe)],
                        agg_hbm.at[core].at[pl.ds(sub * stripe, stripe)])

    return ak(nodes_pad, bsrc4, bdst4, cnts, zeros_rows)


def _gates(agg_pad, h_in, bias, w6, b3, n, d):
    blk = 1000
    nb = n // (2 * blk)

    def body(agg_ref, h_ref, bias_ref, w6_ref, b3_ref, out_ref):
        agg = agg_ref[0] + bias_ref[...]
        m = jnp.max(agg, axis=-1, keepdims=True)
        e = jnp.exp(agg - m)
        a = e / jnp.sum(e, axis=-1, keepdims=True)
        h = h_ref[...]

        def mm(x, w):
            return jnp.dot(x, w, preferred_element_type=jnp.float32)

        r = jax.nn.sigmoid(mm(a, w6_ref[0]) + mm(h, w6_ref[1]) + b3_ref[0])
        z = jax.nn.sigmoid(mm(a, w6_ref[2]) + mm(h, w6_ref[3]) + b3_ref[1])
        hh = jnp.tanh(mm(a, w6_ref[4]) + mm(r * h, w6_ref[5]) + b3_ref[2])
        out_ref[...] = (1.0 - z) * h + z * hh

    return pl.pallas_call(
        body,
        grid=(n // blk,),
        in_specs=[
            pl.BlockSpec((1, blk, d), lambda i, nb=nb: (i // nb, i % nb, 0)),
            pl.BlockSpec((blk, d), lambda i: (i, 0)),
            pl.BlockSpec((1, d), lambda i: (0, 0)),
            pl.BlockSpec((6, d, d), lambda i: (0, 0, 0)),
            pl.BlockSpec((3, d), lambda i: (0, 0)),
        ],
        out_specs=pl.BlockSpec((blk, d), lambda i: (i, 0)),
        out_shape=jax.ShapeDtypeStruct((n, d), jnp.float32),
    )(agg_pad, h_in, bias, w6, b3)


def kernel(nodes_ft, adj_list, bias, Wr, br, Wz, bz, Wt, bt):
    n, d = nodes_ft.shape
    e = adj_list.shape[1]
    half = n // 2
    hrows = _NS * (-(-(half + _C // 2) // (8 * _NS)) * 8)
    n_dummy = hrows - half

    epw = -(-e // _NW)
    epw_pad = -(-epw // _C) * _C
    pw = epw_pad - epw
    dst = adj_list[0].reshape(_NW, epw)
    src = adj_list[1].reshape(_NW, epw)
    pad_d = jnp.broadcast_to(n + (jnp.arange(pw, dtype=jnp.int32) % n_dummy),
                             (_NW, pw))
    pad_s = jnp.zeros((_NW, pw), jnp.int32)
    src_wk = jnp.concatenate([src, pad_s], axis=1).reshape(_NW, -1, _C)
    dst_wk = jnp.concatenate([dst, pad_d], axis=1).reshape(_NW, -1, _C)

    bsrc, bdst, cnts = _partition_edges(src_wk, dst_wk, half)
    bsrc4 = bsrc.reshape(_NW, 4, _CAPC, _C)
    bdst4 = bdst.reshape(_NW, 4, _CAPC, _C)

    nodes_pad = jnp.pad(nodes_ft, ((0, 2 * hrows - n), (0, 0)))
    zeros_rows = jnp.zeros((hrows // _NS, d), jnp.float32)
    agg_pad = _aggregate(nodes_pad, bsrc4, bdst4, cnts, zeros_rows,
                         half, hrows, d)

    w6 = jnp.stack([Wr[:, :d].T, Wr[:, d:].T,
                    Wz[:, :d].T, Wz[:, d:].T,
                    Wt[:, :d].T, Wt[:, d:].T])
    b3 = jnp.stack([br, bz, bt])
    return _gates(agg_pad, nodes_ft, bias, w6, b3, n, d)
